# baseline (speedup 1.0000x reference)
import functools

import jax
import jax.numpy as jnp
from jax import lax
from jax.experimental import pallas as pl
from jax.experimental.pallas import tpu as pltpu
from jax.experimental.pallas import tpu_sc as plsc

N = 10000
E = 320000
D = 128

NC = 2
NS = 16
NW = NC * NS

CHUNK = 128
EPW = E // NW
NPAIR = -(-EPW // (2 * CHUNK))
NCHUNK = 2 * NPAIR
EPW_PAD = NCHUNK * CHUNK
E_PAD = EPW_PAD * NW

ROWS_PER_TILE = 640
N_PAD = ROWS_PER_TILE * NS


def _sc_aggregate_body(with_deg, x_hbm, src_hbm, dst_hbm, zeros_hbm, ones_hbm,
                       agg_out, deg_out,
                       idx_src, dstb0, dstb1, rows0, rows1, ones_v, acc, dacc,
                       sem, sem_i):
  c = lax.axis_index("c")
  s = lax.axis_index("s")
  wid = c * NS + s

  pltpu.sync_copy(zeros_hbm, rows0)
  base = s * ROWS_PER_TILE
  for i in range(ROWS_PER_TILE // CHUNK):
    pltpu.sync_copy(rows0, acc.at[pl.ds(base + i * CHUNK, CHUNK)])
  if with_deg:
    for i in range(ROWS_PER_TILE // D):
      pltpu.sync_copy(rows0.at[0], dacc.at[pl.ds(base + i * D, D)])
    pltpu.sync_copy(ones_hbm, ones_v)
  pltpu.sync_copy(src_hbm.at[wid], idx_src)
  plsc.subcore_barrier()

  def scatter(rows, dstb):
    pltpu.sync_copy(rows, acc.at[dstb.at[0]], add=True)
    if with_deg:
      pltpu.sync_copy(ones_v, dacc.at[dstb.at[0]], add=True)

  pltpu.async_copy(x_hbm.at[idx_src.at[0]], rows0, sem)
  pltpu.async_copy(dst_hbm.at[wid, 0], dstb0.at[0], sem_i)

  def pair_step(p, carry):
    i0 = 2 * p
    pltpu.make_async_copy(x_hbm.at[idx_src.at[i0]], rows0, sem).wait()
    pltpu.async_copy(x_hbm.at[idx_src.at[i0 + 1]], rows1, sem)
    pltpu.make_async_copy(dst_hbm.at[wid, i0], dstb0.at[0], sem_i).wait()
    pltpu.async_copy(dst_hbm.at[wid, i0 + 1], dstb1.at[0], sem_i)
    scatter(rows0, dstb0)
    pltpu.make_async_copy(x_hbm.at[idx_src.at[i0 + 1]], rows1, sem).wait()
    pltpu.make_async_copy(dst_hbm.at[wid, i0 + 1], dstb1.at[0], sem_i).wait()

    @pl.when(p < NPAIR - 1)
    def _():
      pltpu.async_copy(x_hbm.at[idx_src.at[i0 + 2]], rows0, sem)
      pltpu.async_copy(dst_hbm.at[wid, i0 + 2], dstb0.at[0], sem_i)

    scatter(rows1, dstb1)
    return carry

  lax.fori_loop(0, NPAIR, pair_step, 0)

  plsc.subcore_barrier()
  pltpu.sync_copy(acc.at[pl.ds(base, ROWS_PER_TILE)],
                  agg_out.at[c, pl.ds(base, ROWS_PER_TILE)])
  if with_deg:
    pltpu.sync_copy(dacc.at[pl.ds(base, ROWS_PER_TILE)],
                    deg_out.at[c, pl.ds(base, ROWS_PER_TILE)])


def _make_sc_aggregate(with_deg):
  out_type = [jax.ShapeDtypeStruct((NC, N_PAD, D), jnp.float32)]
  if with_deg:
    out_type.append(jax.ShapeDtypeStruct((NC, N_PAD), jnp.float32))
  else:
    out_type.append(jax.ShapeDtypeStruct((1, 1), jnp.float32))
  return functools.partial(
      pl.kernel,
      out_type=tuple(out_type),
      mesh=plsc.VectorSubcoreMesh(core_axis_name="c", subcore_axis_name="s"),
      scratch_types=[
          pltpu.VMEM((NCHUNK, CHUNK), jnp.int32),
          pltpu.VMEM((1, CHUNK), jnp.int32),
          pltpu.VMEM((1, CHUNK), jnp.int32),
          pltpu.VMEM((CHUNK, D), jnp.float32),
          pltpu.VMEM((CHUNK, D), jnp.float32),
          pltpu.VMEM((CHUNK,), jnp.float32),
          pltpu.VMEM_SHARED((N_PAD, D), jnp.float32),
          pltpu.VMEM_SHARED((N_PAD,), jnp.float32),
          pltpu.SemaphoreType.DMA,
          pltpu.SemaphoreType.DMA,
      ],
  )(functools.partial(_sc_aggregate_body, with_deg))


_sc_aggregate_deg = _make_sc_aggregate(True)
_sc_aggregate_nodeg = _make_sc_aggregate(False)


def _tc_dense_body(x_ref, a0, a1, d0, d1, w, b, g, be, o):
  agg = a0[...] + a1[...]
  deg = jnp.maximum(d0[...] + d1[...], 1.0)
  agg = agg / deg
  h = jnp.dot(x_ref[...], w[:D, :], preferred_element_type=jnp.float32)
  h = h + jnp.dot(agg, w[D:, :], preferred_element_type=jnp.float32)
  h = jnp.maximum(h + b[...], 0.0)
  mean = jnp.mean(h, axis=0, keepdims=True)
  zm = h - mean
  var = jnp.mean(zm * zm, axis=0, keepdims=True)
  hn = zm * lax.rsqrt(var + 1e-5) * g[...] + be[...]
  nrm = jnp.sqrt(jnp.sum(hn * hn, axis=1, keepdims=True))
  o[...] = hn / (nrm + 1e-6)


def _tc_dense(x, aggp, degp, w, b, g, be):
  return pl.pallas_call(
      _tc_dense_body,
      out_shape=jax.ShapeDtypeStruct((N, D), jnp.float32),
  )(x, aggp[0, :N], aggp[1, :N],
    degp[0, :N].reshape(N, 1), degp[1, :N].reshape(N, 1),
    w, b.reshape(1, D), g.reshape(1, D), be.reshape(1, D))


def kernel(features, edge_index, W1, b1, g1, be1, W2, b2, g2, be2):
  src = edge_index[0]
  dst = edge_index[1]
  pad = E_PAD - E
  src_p = jnp.concatenate([src, jnp.zeros((pad,), jnp.int32)]).reshape(NW, NCHUNK, CHUNK)
  dst_p = jnp.concatenate([dst, jnp.full((pad,), N, jnp.int32)]).reshape(NW, NCHUNK, CHUNK)
  zeros = jnp.zeros((CHUNK, D), jnp.float32)
  ones = jnp.ones((CHUNK,), jnp.float32)

  agg1, deg = _sc_aggregate_deg(features, src_p, dst_p, zeros, ones)
  h1 = _tc_dense(features, agg1, deg, W1, b1, g1, be1)
  agg2, _ = _sc_aggregate_nodeg(h1, src_p, dst_p, zeros, ones)
  h2 = _tc_dense(h1, agg2, deg, W2, b2, g2, be2)
  return h2


# speedup vs baseline: 1.0952x; 1.0952x over previous
import functools

import jax
import jax.numpy as jnp
from jax import lax
from jax.experimental import pallas as pl
from jax.experimental.pallas import tpu as pltpu
from jax.experimental.pallas import tpu_sc as plsc

N = 10000
E = 320000
D = 128

NC = 2
NS = 16
NW = NC * NS
LANES = 16

CHUNK = 128
EPW = E // NW
NPAIR = -(-EPW // (2 * CHUNK))
NCHUNK = 2 * NPAIR
EPW_PAD = NCHUNK * CHUNK
E_PAD = EPW_PAD * NW

PACK = 1 << 14

ROWS_PER_TILE = 640
N_PAD = ROWS_PER_TILE * NS


def _sc_aggregate_body(with_deg, x_hbm, pidx_hbm, zeros_hbm, ones_hbm,
                       agg_out, deg_out,
                       pidx, srcb, dstb, rows0, rows1, ones_v, acc, dacc,
                       sem0, sem1):
  c = lax.axis_index("c")
  s = lax.axis_index("s")
  wid = c * NS + s

  pltpu.sync_copy(zeros_hbm, rows0)
  base = s * ROWS_PER_TILE
  for i in range(ROWS_PER_TILE // CHUNK):
    pltpu.sync_copy(rows0, acc.at[pl.ds(base + i * CHUNK, CHUNK)])
  if with_deg:
    for i in range(ROWS_PER_TILE // D):
      pltpu.sync_copy(rows0.at[0], dacc.at[pl.ds(base + i * D, D)])
    pltpu.sync_copy(ones_hbm, ones_v)
  pltpu.sync_copy(pidx_hbm.at[wid], pidx)
  plsc.subcore_barrier()

  def unpack(i, b):
    for j in range(CHUNK // LANES):
      v = pidx[i, pl.ds(j * LANES, LANES)]
      srcb[b, pl.ds(j * LANES, LANES)] = jax.lax.shift_right_logical(v, PACK.bit_length() - 1)
      dstb[b, pl.ds(j * LANES, LANES)] = jax.lax.bitwise_and(v, PACK - 1)

  def scatter(rows, b):
    pltpu.sync_copy(rows, acc.at[dstb.at[b]], add=True)
    if with_deg:
      pltpu.sync_copy(ones_v, dacc.at[dstb.at[b]], add=True)

  unpack(0, 0)
  pltpu.async_copy(x_hbm.at[srcb.at[0]], rows0, sem0)

  def pair_step(p, carry):
    i0 = 2 * p
    unpack(i0 + 1, 1)
    pltpu.make_async_copy(x_hbm.at[srcb.at[0]], rows0, sem0).wait()
    pltpu.async_copy(x_hbm.at[srcb.at[1]], rows1, sem1)
    scatter(rows0, 0)

    @pl.when(p < NPAIR - 1)
    def _():
      unpack(i0 + 2, 0)

    pltpu.make_async_copy(x_hbm.at[srcb.at[1]], rows1, sem1).wait()

    @pl.when(p < NPAIR - 1)
    def _():
      pltpu.async_copy(x_hbm.at[srcb.at[0]], rows0, sem0)

    scatter(rows1, 1)
    return carry

  lax.fori_loop(0, NPAIR, pair_step, 0)

  plsc.subcore_barrier()
  pltpu.sync_copy(acc.at[pl.ds(base, ROWS_PER_TILE)],
                  agg_out.at[c, pl.ds(base, ROWS_PER_TILE)])
  if with_deg:
    pltpu.sync_copy(dacc.at[pl.ds(base, ROWS_PER_TILE)],
                    deg_out.at[c, pl.ds(base, ROWS_PER_TILE)])


def _make_sc_aggregate(with_deg):
  out_type = [jax.ShapeDtypeStruct((NC, N_PAD, D), jnp.float32)]
  if with_deg:
    out_type.append(jax.ShapeDtypeStruct((NC, N_PAD), jnp.float32))
  else:
    out_type.append(jax.ShapeDtypeStruct((1, 1), jnp.float32))
  return functools.partial(
      pl.kernel,
      out_type=tuple(out_type),
      mesh=plsc.VectorSubcoreMesh(core_axis_name="c", subcore_axis_name="s"),
      scratch_types=[
          pltpu.VMEM((NCHUNK, CHUNK), jnp.int32),
          pltpu.VMEM((2, CHUNK), jnp.int32),
          pltpu.VMEM((2, CHUNK), jnp.int32),
          pltpu.VMEM((CHUNK, D), jnp.float32),
          pltpu.VMEM((CHUNK, D), jnp.float32),
          pltpu.VMEM((CHUNK,), jnp.float32),
          pltpu.VMEM_SHARED((N_PAD, D), jnp.float32),
          pltpu.VMEM_SHARED((N_PAD,), jnp.float32),
          pltpu.SemaphoreType.DMA,
          pltpu.SemaphoreType.DMA,
      ],
  )(functools.partial(_sc_aggregate_body, with_deg))


_sc_aggregate_deg = _make_sc_aggregate(True)
_sc_aggregate_nodeg = _make_sc_aggregate(False)


def _tc_dense_body(x_ref, a0, a1, d0, d1, w, b, g, be, o):
  agg = a0[...] + a1[...]
  deg = jnp.maximum(d0[...] + d1[...], 1.0)
  agg = agg / deg
  h = jnp.dot(x_ref[...], w[:D, :], preferred_element_type=jnp.float32)
  h = h + jnp.dot(agg, w[D:, :], preferred_element_type=jnp.float32)
  h = jnp.maximum(h + b[...], 0.0)
  mean = jnp.mean(h, axis=0, keepdims=True)
  zm = h - mean
  var = jnp.mean(zm * zm, axis=0, keepdims=True)
  hn = zm * lax.rsqrt(var + 1e-5) * g[...] + be[...]
  nrm = jnp.sqrt(jnp.sum(hn * hn, axis=1, keepdims=True))
  o[...] = hn / (nrm + 1e-6)


def _tc_dense(x, aggp, degp, w, b, g, be):
  return pl.pallas_call(
      _tc_dense_body,
      out_shape=jax.ShapeDtypeStruct((N, D), jnp.float32),
  )(x, aggp[0, :N], aggp[1, :N],
    degp[0, :N].reshape(N, 1), degp[1, :N].reshape(N, 1),
    w, b.reshape(1, D), g.reshape(1, D), be.reshape(1, D))


def kernel(features, edge_index, W1, b1, g1, be1, W2, b2, g2, be2):
  src = edge_index[0]
  dst = edge_index[1]
  pad = E_PAD - E
  packed = src * PACK + dst
  packed = jnp.concatenate([packed, jnp.full((pad,), N, jnp.int32)])
  pidx = packed.reshape(NW, NCHUNK, CHUNK)
  zeros = jnp.zeros((CHUNK, D), jnp.float32)
  ones = jnp.ones((CHUNK,), jnp.float32)

  agg1, deg = _sc_aggregate_deg(features, pidx, zeros, ones)
  h1 = _tc_dense(features, agg1, deg, W1, b1, g1, be1)
  agg2, _ = _sc_aggregate_nodeg(h1, pidx, zeros, ones)
  h2 = _tc_dense(h1, agg2, deg, W2, b2, g2, be2)
  return h2


# speedup vs baseline: 1.0967x; 1.0014x over previous
import functools

import jax
import jax.numpy as jnp
from jax import lax
from jax.experimental import pallas as pl
from jax.experimental.pallas import tpu as pltpu
from jax.experimental.pallas import tpu_sc as plsc

N = 10000
E = 320000
D = 128

NC = 2
NS = 16
NW = NC * NS
LANES = 16

CHUNK = 128
EPW = E // NW
NPAIR = -(-EPW // (2 * CHUNK))
NCHUNK = 2 * NPAIR
EPW_PAD = NCHUNK * CHUNK
E_PAD = EPW_PAD * NW

PACK = 1 << 14

ROWS_PER_TILE = 640
N_PAD = ROWS_PER_TILE * NS


def _sc_aggregate_body(with_deg, x_hbm, pidx_hbm, zeros_hbm, ones_hbm,
                       agg_out, deg_out,
                       pidx, srcb, dstb, rows0, rows1, ones_v, acc, dacc,
                       sem0, sem1):
  c = lax.axis_index("c")
  s = lax.axis_index("s")
  wid = c * NS + s

  pltpu.sync_copy(zeros_hbm, rows0)
  base = s * ROWS_PER_TILE
  for i in range(ROWS_PER_TILE // CHUNK):
    pltpu.sync_copy(rows0, acc.at[pl.ds(base + i * CHUNK, CHUNK)])
  if with_deg:
    for i in range(ROWS_PER_TILE // D):
      pltpu.sync_copy(rows0.at[0], dacc.at[pl.ds(base + i * D, D)])
    pltpu.sync_copy(ones_hbm, ones_v)
  pltpu.sync_copy(pidx_hbm.at[wid], pidx)
  plsc.subcore_barrier()

  def unpack(i, b):
    for j in range(CHUNK // LANES):
      v = pidx[i, pl.ds(j * LANES, LANES)]
      srcb[b, pl.ds(j * LANES, LANES)] = jax.lax.shift_right_logical(v, PACK.bit_length() - 1)
      dstb[b, pl.ds(j * LANES, LANES)] = jax.lax.bitwise_and(v, PACK - 1)

  def scatter(rows, b):
    pltpu.sync_copy(rows, acc.at[dstb.at[b]], add=True)
    if with_deg:
      pltpu.sync_copy(ones_v, dacc.at[dstb.at[b]], add=True)

  unpack(0, 0)
  pltpu.async_copy(x_hbm.at[srcb.at[0]], rows0, sem0)

  def pair_step(p, carry):
    i0 = 2 * p
    unpack(i0 + 1, 1)
    pltpu.make_async_copy(x_hbm.at[srcb.at[0]], rows0, sem0).wait()
    pltpu.async_copy(x_hbm.at[srcb.at[1]], rows1, sem1)
    scatter(rows0, 0)

    @pl.when(p < NPAIR - 1)
    def _():
      unpack(i0 + 2, 0)

    pltpu.make_async_copy(x_hbm.at[srcb.at[1]], rows1, sem1).wait()

    @pl.when(p < NPAIR - 1)
    def _():
      pltpu.async_copy(x_hbm.at[srcb.at[0]], rows0, sem0)

    scatter(rows1, 1)
    return carry

  lax.fori_loop(0, NPAIR, pair_step, 0)

  plsc.subcore_barrier()
  pltpu.sync_copy(acc.at[pl.ds(base, ROWS_PER_TILE)],
                  agg_out.at[c, pl.ds(base, ROWS_PER_TILE)])
  if with_deg:
    pltpu.sync_copy(dacc.at[pl.ds(base, ROWS_PER_TILE)],
                    deg_out.at[c, pl.ds(base, ROWS_PER_TILE)])


def _make_sc_aggregate(with_deg):
  out_type = [jax.ShapeDtypeStruct((NC, N_PAD, D), jnp.float32)]
  if with_deg:
    out_type.append(jax.ShapeDtypeStruct((NC, N_PAD), jnp.float32))
  else:
    out_type.append(jax.ShapeDtypeStruct((1, 1), jnp.float32))
  return functools.partial(
      pl.kernel,
      out_type=tuple(out_type),
      mesh=plsc.VectorSubcoreMesh(core_axis_name="c", subcore_axis_name="s"),
      scratch_types=[
          pltpu.VMEM((NCHUNK, CHUNK), jnp.int32),
          pltpu.VMEM((2, CHUNK), jnp.int32),
          pltpu.VMEM((2, CHUNK), jnp.int32),
          pltpu.VMEM((CHUNK, D), jnp.float32),
          pltpu.VMEM((CHUNK, D), jnp.float32),
          pltpu.VMEM((CHUNK,), jnp.float32),
          pltpu.VMEM_SHARED((N_PAD, D), jnp.float32),
          pltpu.VMEM_SHARED((N_PAD,), jnp.float32),
          pltpu.SemaphoreType.DMA,
          pltpu.SemaphoreType.DMA,
      ],
  )(functools.partial(_sc_aggregate_body, with_deg))


_sc_aggregate_deg = _make_sc_aggregate(True)
_sc_aggregate_nodeg = _make_sc_aggregate(False)


def _tc_dense_body(x_ref, a0, a1, d0, d1, w, b, g, be, o):
  agg = a0[...] + a1[...]
  deg = jnp.maximum(d0[...] + d1[...], 1.0)
  agg = agg / deg
  h = jnp.dot(x_ref[...], w[:D, :], preferred_element_type=jnp.float32)
  h = h + jnp.dot(agg, w[D:, :], preferred_element_type=jnp.float32)
  h = jnp.maximum(h + b[...], 0.0)
  mean = jnp.mean(h, axis=0, keepdims=True)
  zm = h - mean
  var = jnp.mean(zm * zm, axis=0, keepdims=True)
  hn = zm * lax.rsqrt(var + 1e-5) * g[...] + be[...]
  nrm = jnp.sqrt(jnp.sum(hn * hn, axis=1, keepdims=True))
  o[...] = hn / (nrm + 1e-6)


def _tc_dense(x, aggp, degp, w, b, g, be):
  return pl.pallas_call(
      _tc_dense_body,
      out_shape=jax.ShapeDtypeStruct((N, D), jnp.float32),
  )(x, aggp[0, :N], aggp[1, :N],
    degp[0, :N].reshape(N, 1), degp[1, :N].reshape(N, 1),
    w, b.reshape(1, D), g.reshape(1, D), be.reshape(1, D))


def kernel(features, edge_index, W1, b1, g1, be1, W2, b2, g2, be2):
  src = edge_index[0]
  dst = edge_index[1]
  pad = E_PAD - E
  packed = src * PACK + dst
  dummy_dst = N + jnp.arange(pad, dtype=jnp.int32) % (N_PAD - N)
  packed = jnp.concatenate([packed, dummy_dst])
  pidx = packed.reshape(NW, NCHUNK, CHUNK)
  zeros = jnp.zeros((CHUNK, D), jnp.float32)
  ones = jnp.ones((CHUNK,), jnp.float32)

  agg1, deg = _sc_aggregate_deg(features, pidx, zeros, ones)
  h1 = _tc_dense(features, agg1, deg, W1, b1, g1, be1)
  agg2, _ = _sc_aggregate_nodeg(h1, pidx, zeros, ones)
  h2 = _tc_dense(h1, agg2, deg, W2, b2, g2, be2)
  return h2
